# baseline (speedup 1.0000x reference)
import jax
from jax.experimental import pallas as pl
from jax.experimental.pallas import tpu as pltpu


def _identity_copy(x_ref, e_ref, x_out_ref, e_out_ref, sem_x, sem_e):
    copy_x = pltpu.make_async_copy(x_ref, x_out_ref, sem_x)
    copy_e = pltpu.make_async_copy(e_ref, e_out_ref, sem_e)
    copy_x.start()
    copy_e.start()
    copy_x.wait()
    copy_e.wait()


def kernel(x, edge_index, edge_attr):
    del edge_index
    x_out, e_out = pl.pallas_call(
        _identity_copy,
        out_shape=(
            jax.ShapeDtypeStruct(x.shape, x.dtype),
            jax.ShapeDtypeStruct(edge_attr.shape, edge_attr.dtype),
        ),
        in_specs=[
            pl.BlockSpec(memory_space=pl.ANY),
            pl.BlockSpec(memory_space=pl.ANY),
        ],
        out_specs=(
            pl.BlockSpec(memory_space=pl.ANY),
            pl.BlockSpec(memory_space=pl.ANY),
        ),
        scratch_shapes=[pltpu.SemaphoreType.DMA, pltpu.SemaphoreType.DMA],
    )(x, edge_attr)
    return (x_out, e_out)


# speedup vs baseline: 17.5752x; 17.5752x over previous
import jax
import jax.numpy as jnp
from jax.experimental import pallas as pl


def _copy_body(x_ref, e_ref, x_out_ref, e_out_ref):
    x_out_ref[...] = x_ref[...]
    e_out_ref[...] = e_ref[...]


def kernel(x, edge_index, edge_attr):
    del edge_index
    n_nodes, d_feat = x.shape
    n_edges, d_edge = edge_attr.shape
    e2 = edge_attr.reshape(-1, 128)
    e_rows = e2.shape[0]

    grid = 10
    bx = n_nodes // grid
    be = e_rows // grid

    x_out, e_out = pl.pallas_call(
        _copy_body,
        grid=(grid,),
        out_shape=(
            jax.ShapeDtypeStruct(x.shape, x.dtype),
            jax.ShapeDtypeStruct(e2.shape, e2.dtype),
        ),
        in_specs=[
            pl.BlockSpec((bx, d_feat), lambda i: (i, 0)),
            pl.BlockSpec((be, 128), lambda i: (i, 0)),
        ],
        out_specs=(
            pl.BlockSpec((bx, d_feat), lambda i: (i, 0)),
            pl.BlockSpec((be, 128), lambda i: (i, 0)),
        ),
    )(x, e2)
    return (x_out, e_out.reshape(n_edges, d_edge))


# speedup vs baseline: 19.3063x; 1.0985x over previous
import jax
import jax.numpy as jnp
from jax.experimental import pallas as pl


def _copy_body(x_ref, e_ref, x_out_ref, e_out_ref):
    x_out_ref[...] = x_ref[...]
    e_out_ref[...] = e_ref[...]


def kernel(x, edge_index, edge_attr):
    del edge_index
    n_nodes, d_feat = x.shape
    n_edges, d_edge = edge_attr.shape

    grid = 25
    bx = n_nodes // grid
    be = n_edges // grid

    x_out, e_out = pl.pallas_call(
        _copy_body,
        grid=(grid,),
        out_shape=(
            jax.ShapeDtypeStruct(x.shape, x.dtype),
            jax.ShapeDtypeStruct(edge_attr.shape, edge_attr.dtype),
        ),
        in_specs=[
            pl.BlockSpec((bx, d_feat), lambda i: (i, 0)),
            pl.BlockSpec((be, d_edge), lambda i: (i, 0)),
        ],
        out_specs=(
            pl.BlockSpec((bx, d_feat), lambda i: (i, 0)),
            pl.BlockSpec((be, d_edge), lambda i: (i, 0)),
        ),
    )(x, edge_attr)
    return (x_out, e_out)


# speedup vs baseline: 174.7629x; 9.0521x over previous
import jax
import jax.numpy as jnp
from jax.experimental import pallas as pl


def _copy_body(x_ref, x_out_ref):
    x_out_ref[...] = x_ref[...]


def kernel(x, edge_index, edge_attr):
    del edge_index
    n_nodes, d_feat = x.shape
    grid = 25
    bx = n_nodes // grid
    x_out = pl.pallas_call(
        _copy_body,
        grid=(grid,),
        out_shape=jax.ShapeDtypeStruct(x.shape, x.dtype),
        in_specs=[pl.BlockSpec((bx, d_feat), lambda i: (i, 0))],
        out_specs=pl.BlockSpec((bx, d_feat), lambda i: (i, 0)),
    )(x)
    return (x_out, edge_attr)
